# baseline (speedup 1.0000x reference)
import functools

import jax
import jax.numpy as jnp
from jax.experimental import pallas as pl

P = 512
H = 64
E = 32
N = P * P


def _lstm_block_kernel(x_ref, y_ref, nei_ref, ht_ref, ct_ref,
                       w0_ref, w1_ref, bemb_ref, wih_ref, whh_ref, bias_ref,
                       ho_ref, co_ref):
    g3 = ht_ref.shape[0]
    rows = g3 * P
    x = x_ref[...]
    y = y_ref[...]
    ht = ht_ref[...].reshape(rows, H)
    ct = ct_ref[...].reshape(rows, H)
    emb = jnp.maximum(x * w0_ref[...] + y * w1_ref[...] + bemb_ref[...], 0.0)
    gates = (jnp.dot(emb, wih_ref[...], preferred_element_type=jnp.float32)
             + jnp.dot(ht, whh_ref[...], preferred_element_type=jnp.float32)
             + bias_ref[...])
    i = jax.nn.sigmoid(gates[:, 0 * H:1 * H])
    f = jax.nn.sigmoid(gates[:, 1 * H:2 * H])
    g = jnp.tanh(gates[:, 2 * H:3 * H])
    o = jax.nn.sigmoid(gates[:, 3 * H:4 * H])
    c_new = f * ct + i * g
    h_new = o * jnp.tanh(c_new)
    m = nei_ref[...] > 0
    ho_ref[...] = jnp.where(m, h_new, ht).reshape(g3, P, H)
    co_ref[...] = jnp.where(m, c_new, ct).reshape(g3, P, H)


@functools.partial(jax.jit, static_argnames=("g3",))
def _run(corr_index, rela_ht, rela_ct, nei_index,
         W_emb, b_emb, W_ih, W_hh, b_ih, b_hh, g3=2):
    x = corr_index[:, :, 0].reshape(N, 1)
    y = corr_index[:, :, 1].reshape(N, 1)
    nei = nei_index.reshape(N, 1)
    w0 = W_emb[:, 0].reshape(1, E)
    w1 = W_emb[:, 1].reshape(1, E)
    bemb = b_emb.reshape(1, E)
    wih = W_ih.T
    whh = W_hh.T
    bias = (b_ih + b_hh).reshape(1, 4 * H)

    rows = g3 * P
    grid = (P // g3,)
    col_spec = pl.BlockSpec((rows, 1), lambda i: (i, 0))
    st_spec = pl.BlockSpec((g3, P, H), lambda i: (i, 0, 0))
    full_spec = lambda r, c: pl.BlockSpec((r, c), lambda i: (0, 0))
    ho, co = pl.pallas_call(
        _lstm_block_kernel,
        grid=grid,
        in_specs=[
            col_spec,
            col_spec,
            col_spec,
            st_spec,
            st_spec,
            full_spec(1, E),
            full_spec(1, E),
            full_spec(1, E),
            full_spec(E, 4 * H),
            full_spec(H, 4 * H),
            full_spec(1, 4 * H),
        ],
        out_specs=[st_spec, st_spec],
        out_shape=[
            jax.ShapeDtypeStruct((P, P, H), jnp.float32),
            jax.ShapeDtypeStruct((P, P, H), jnp.float32),
        ],
    )(x, y, nei, rela_ht, rela_ct, w0, w1, bemb, wih, whh, bias)
    return ho, co


def kernel(corr_index, rela_ht, rela_ct, nei_index,
           W_emb, b_emb, W_ih, W_hh, b_ih, b_hh):
    return _run(corr_index, rela_ht, rela_ct, nei_index,
                W_emb, b_emb, W_ih, W_hh, b_ih, b_hh)


# speedup vs baseline: 5.2617x; 5.2617x over previous
import functools

import jax
import jax.numpy as jnp
from jax.experimental import pallas as pl

P = 512
H = 64
E = 32
N = P * P


def _lstm_slab_kernel(corr_ref, nei_ref, ht_ref, ct_ref,
                      wemb_ref, bemb_ref, wih_ref, whh_ref, bias_ref,
                      ho_ref, co_ref):
    g = ht_ref.shape[0]
    for j in range(g):
        corr = corr_ref[j]
        ht = ht_ref[j]
        ct = ct_ref[j]
        emb = jnp.maximum(
            jnp.dot(wemb_ref[...], corr, preferred_element_type=jnp.float32)
            + bemb_ref[...], 0.0)
        gates = (jnp.dot(wih_ref[...], emb, preferred_element_type=jnp.float32)
                 + jnp.dot(whh_ref[...], ht, preferred_element_type=jnp.float32)
                 + bias_ref[...])
        i = jax.nn.sigmoid(gates[0 * H:1 * H, :])
        f = jax.nn.sigmoid(gates[1 * H:2 * H, :])
        gg = jnp.tanh(gates[2 * H:3 * H, :])
        o = jax.nn.sigmoid(gates[3 * H:4 * H, :])
        c_new = f * ct + i * gg
        h_new = o * jnp.tanh(c_new)
        m = nei_ref[j] > 0
        ho_ref[j] = jnp.where(m, h_new, ht)
        co_ref[j] = jnp.where(m, c_new, ct)


@functools.partial(jax.jit, static_argnames=("g",))
def _run(corr_index, rela_ht, rela_ct, nei_index,
         W_emb, b_emb, W_ih, W_hh, b_ih, b_hh, g=4):
    corr_t = jnp.transpose(corr_index, (0, 2, 1))
    nei3 = nei_index.reshape(P, 1, P)
    ht_t = jnp.transpose(rela_ht, (0, 2, 1))
    ct_t = jnp.transpose(rela_ct, (0, 2, 1))
    bemb = b_emb.reshape(E, 1)
    bias = (b_ih + b_hh).reshape(4 * H, 1)

    grid = (P // g,)
    corr_spec = pl.BlockSpec((g, 2, P), lambda i: (i, 0, 0))
    nei_spec = pl.BlockSpec((g, 1, P), lambda i: (i, 0, 0))
    st_spec = pl.BlockSpec((g, H, P), lambda i: (i, 0, 0))
    full_spec = lambda r, c: pl.BlockSpec((r, c), lambda i: (0, 0))
    ho, co = pl.pallas_call(
        _lstm_slab_kernel,
        grid=grid,
        in_specs=[
            corr_spec,
            nei_spec,
            st_spec,
            st_spec,
            full_spec(E, 2),
            full_spec(E, 1),
            full_spec(4 * H, E),
            full_spec(4 * H, H),
            full_spec(4 * H, 1),
        ],
        out_specs=[st_spec, st_spec],
        out_shape=[
            jax.ShapeDtypeStruct((P, H, P), jnp.float32),
            jax.ShapeDtypeStruct((P, H, P), jnp.float32),
        ],
    )(corr_t, nei3, ht_t, ct_t, W_emb, bemb, W_ih, W_hh, bias)
    return jnp.transpose(ho, (0, 2, 1)), jnp.transpose(co, (0, 2, 1))


def kernel(corr_index, rela_ht, rela_ct, nei_index,
           W_emb, b_emb, W_ih, W_hh, b_ih, b_hh):
    return _run(corr_index, rela_ht, rela_ct, nei_index,
                W_emb, b_emb, W_ih, W_hh, b_ih, b_hh)
